# speedup vs baseline: 7.0881x; 7.0881x over previous
import jax
import jax.numpy as jnp
from jax.experimental import pallas as pl
from jax.experimental.pallas import tpu as pltpu

N = 100000
C = 128
B = 64
EPS = 1e-05
BLK = 2000
NB = N // BLK


def _stats_kernel(x_ref, ids_ref, w_ref, b_ref, ms_ref, a_out, bo_out,
                  s1_ref, s2_ref, cnt_ref):
    i = pl.program_id(0)

    @pl.when(i == 0)
    def _init():
        s1_ref[...] = jnp.zeros_like(s1_ref)
        s2_ref[...] = jnp.zeros_like(s2_ref)
        cnt_ref[...] = jnp.zeros_like(cnt_ref)

    x = x_ref[...]
    ids = ids_ref[0]
    onehot_t = (jax.lax.broadcasted_iota(jnp.int32, (B, BLK), 0) == ids
                ).astype(jnp.float32)
    s1_ref[...] += jnp.dot(onehot_t, x, preferred_element_type=jnp.float32)
    s2_ref[...] += jnp.dot(onehot_t, x * x,
                           preferred_element_type=jnp.float32)
    cnt_ref[...] += jnp.sum(onehot_t, axis=1, keepdims=True)

    @pl.when(i == NB - 1)
    def _finish():
        cnt = jnp.maximum(cnt_ref[...], 1.0)
        mean = s1_ref[...] / cnt
        m2 = mean * ms_ref[...]
        var = s2_ref[...] / cnt - 2.0 * m2 * mean + m2 * m2
        rstd = jax.lax.rsqrt(var + EPS)
        a = w_ref[...] * rstd
        a_out[...] = a
        bo_out[...] = b_ref[...] - a * m2


def _norm_kernel(x_ref, ids_ref, a_ref, b_ref, o_ref):
    ids = ids_ref[...]
    onehot = (jax.lax.broadcasted_iota(jnp.int32, (BLK, B), 1) == ids
              ).astype(jnp.float32)
    ga = jnp.dot(onehot, a_ref[...], preferred_element_type=jnp.float32)
    gb = jnp.dot(onehot, b_ref[...], preferred_element_type=jnp.float32)
    o_ref[...] = ga * x_ref[...] + gb


@jax.jit
def kernel(x, batch, weight, bias, mean_scale):
    ids = batch.astype(jnp.int32)
    ids3 = ids.reshape(NB, 1, BLK)
    ids2 = ids.reshape(N, 1)
    w2 = weight.reshape(1, C)
    b2 = bias.reshape(1, C)
    ms2 = mean_scale.reshape(1, C)

    a_tab, b_tab = pl.pallas_call(
        _stats_kernel,
        grid=(NB,),
        in_specs=[
            pl.BlockSpec((BLK, C), lambda i: (i, 0)),
            pl.BlockSpec((1, 1, BLK), lambda i: (i, 0, 0)),
            pl.BlockSpec((1, C), lambda i: (0, 0)),
            pl.BlockSpec((1, C), lambda i: (0, 0)),
            pl.BlockSpec((1, C), lambda i: (0, 0)),
        ],
        out_specs=[
            pl.BlockSpec((B, C), lambda i: (0, 0)),
            pl.BlockSpec((B, C), lambda i: (0, 0)),
        ],
        out_shape=[
            jax.ShapeDtypeStruct((B, C), jnp.float32),
            jax.ShapeDtypeStruct((B, C), jnp.float32),
        ],
        scratch_shapes=[
            pltpu.VMEM((B, C), jnp.float32),
            pltpu.VMEM((B, C), jnp.float32),
            pltpu.VMEM((B, 1), jnp.float32),
        ],
    )(x, ids3, w2, b2, ms2)

    out = pl.pallas_call(
        _norm_kernel,
        grid=(NB,),
        in_specs=[
            pl.BlockSpec((BLK, C), lambda i: (i, 0)),
            pl.BlockSpec((BLK, 1), lambda i: (i, 0)),
            pl.BlockSpec((B, C), lambda i: (0, 0)),
            pl.BlockSpec((B, C), lambda i: (0, 0)),
        ],
        out_specs=pl.BlockSpec((BLK, C), lambda i: (i, 0)),
        out_shape=jax.ShapeDtypeStruct((N, C), jnp.float32),
    )(x, ids2, a_tab, b_tab)
    return out


# speedup vs baseline: 8.7078x; 1.2285x over previous
import jax
import jax.numpy as jnp
from jax.experimental import pallas as pl
from jax.experimental.pallas import tpu as pltpu

N = 100000
C = 128
B = 64
EPS = 1e-05
BLK = 4000
NB = N // BLK


def _stats_kernel(x_ref, ids_ref, w_ref, b_ref, ms_ref, a_out, bo_out,
                  s1_ref, s2_ref, cnt_ref):
    i = pl.program_id(0)

    @pl.when(i == 0)
    def _init():
        s1_ref[...] = jnp.zeros_like(s1_ref)
        s2_ref[...] = jnp.zeros_like(s2_ref)
        cnt_ref[...] = jnp.zeros_like(cnt_ref)

    x = x_ref[...]
    ids = ids_ref[0]
    onehot_t = (jax.lax.broadcasted_iota(jnp.int32, (B, BLK), 0) == ids
                ).astype(jnp.float32)
    s1_ref[...] += jnp.dot(onehot_t, x, preferred_element_type=jnp.float32)
    s2_ref[...] += jnp.dot(onehot_t, x * x,
                           preferred_element_type=jnp.float32)
    cnt_ref[...] += jnp.sum(onehot_t, axis=1, keepdims=True)

    @pl.when(i == NB - 1)
    def _finish():
        cnt = jnp.maximum(cnt_ref[...], 1.0)
        mean = s1_ref[...] / cnt
        m2 = mean * ms_ref[...]
        var = s2_ref[...] / cnt - 2.0 * m2 * mean + m2 * m2
        rstd = jax.lax.rsqrt(var + EPS)
        a = w_ref[...] * rstd
        a_out[...] = a
        bo_out[...] = b_ref[...] - a * m2


def _norm_kernel(x_ref, ids_ref, a_ref, b_ref, o_ref):
    ids = ids_ref[...]
    onehot = (jax.lax.broadcasted_iota(jnp.int32, (BLK, B), 1) == ids
              ).astype(jnp.float32)
    ga = jnp.dot(onehot, a_ref[...], preferred_element_type=jnp.float32)
    gb = jnp.dot(onehot, b_ref[...], preferred_element_type=jnp.float32)
    o_ref[...] = ga * x_ref[...] + gb


@jax.jit
def kernel(x, batch, weight, bias, mean_scale):
    ids = batch.astype(jnp.int32)
    ids3 = ids.reshape(NB, 1, BLK)
    ids2 = ids.reshape(N, 1)
    w2 = weight.reshape(1, C)
    b2 = bias.reshape(1, C)
    ms2 = mean_scale.reshape(1, C)

    a_tab, b_tab = pl.pallas_call(
        _stats_kernel,
        grid=(NB,),
        in_specs=[
            pl.BlockSpec((BLK, C), lambda i: (i, 0)),
            pl.BlockSpec((1, 1, BLK), lambda i: (i, 0, 0)),
            pl.BlockSpec((1, C), lambda i: (0, 0)),
            pl.BlockSpec((1, C), lambda i: (0, 0)),
            pl.BlockSpec((1, C), lambda i: (0, 0)),
        ],
        out_specs=[
            pl.BlockSpec((B, C), lambda i: (0, 0)),
            pl.BlockSpec((B, C), lambda i: (0, 0)),
        ],
        out_shape=[
            jax.ShapeDtypeStruct((B, C), jnp.float32),
            jax.ShapeDtypeStruct((B, C), jnp.float32),
        ],
        scratch_shapes=[
            pltpu.VMEM((B, C), jnp.float32),
            pltpu.VMEM((B, C), jnp.float32),
            pltpu.VMEM((B, 1), jnp.float32),
        ],
    )(x, ids3, w2, b2, ms2)

    out = pl.pallas_call(
        _norm_kernel,
        grid=(NB,),
        in_specs=[
            pl.BlockSpec((BLK, C), lambda i: (i, 0)),
            pl.BlockSpec((BLK, 1), lambda i: (i, 0)),
            pl.BlockSpec((B, C), lambda i: (0, 0)),
            pl.BlockSpec((B, C), lambda i: (0, 0)),
        ],
        out_specs=pl.BlockSpec((BLK, C), lambda i: (i, 0)),
        out_shape=jax.ShapeDtypeStruct((N, C), jnp.float32),
    )(x, ids2, a_tab, b_tab)
    return out


# speedup vs baseline: 9.6543x; 1.1087x over previous
import jax
import jax.numpy as jnp
from jax.experimental import pallas as pl
from jax.experimental.pallas import tpu as pltpu

N = 100000
C = 128
B = 64
EPS = 1e-05
BLK = 10000
NB = N // BLK


def _stats_kernel(x_ref, ids_ref, w_ref, b_ref, ms_ref, a_out, bo_out,
                  s1_ref, s2_ref, cnt_ref):
    i = pl.program_id(0)

    @pl.when(i == 0)
    def _init():
        s1_ref[...] = jnp.zeros_like(s1_ref)
        s2_ref[...] = jnp.zeros_like(s2_ref)
        cnt_ref[...] = jnp.zeros_like(cnt_ref)

    x = x_ref[...]
    ids = ids_ref[0]
    onehot_t = (jax.lax.broadcasted_iota(jnp.int32, (B, BLK), 0) == ids
                ).astype(jnp.float32)
    s1_ref[...] += jnp.dot(onehot_t, x, preferred_element_type=jnp.float32)
    s2_ref[...] += jnp.dot(onehot_t, x * x,
                           preferred_element_type=jnp.float32)
    cnt_ref[...] += jnp.sum(onehot_t, axis=1, keepdims=True)

    @pl.when(i == NB - 1)
    def _finish():
        cnt = jnp.maximum(cnt_ref[...], 1.0)
        mean = s1_ref[...] / cnt
        m2 = mean * ms_ref[...]
        var = s2_ref[...] / cnt - 2.0 * m2 * mean + m2 * m2
        rstd = jax.lax.rsqrt(var + EPS)
        a = w_ref[...] * rstd
        a_out[...] = a
        bo_out[...] = b_ref[...] - a * m2


def _norm_kernel(x_ref, ids_ref, a_ref, b_ref, o_ref):
    ids = ids_ref[...]
    onehot = (jax.lax.broadcasted_iota(jnp.int32, (BLK, B), 1) == ids
              ).astype(jnp.float32)
    ga = jnp.dot(onehot, a_ref[...], preferred_element_type=jnp.float32)
    gb = jnp.dot(onehot, b_ref[...], preferred_element_type=jnp.float32)
    o_ref[...] = ga * x_ref[...] + gb


@jax.jit
def kernel(x, batch, weight, bias, mean_scale):
    ids = batch.astype(jnp.int32)
    ids3 = ids.reshape(NB, 1, BLK)
    ids2 = ids.reshape(N, 1)
    w2 = weight.reshape(1, C)
    b2 = bias.reshape(1, C)
    ms2 = mean_scale.reshape(1, C)

    a_tab, b_tab = pl.pallas_call(
        _stats_kernel,
        grid=(NB,),
        in_specs=[
            pl.BlockSpec((BLK, C), lambda i: (i, 0)),
            pl.BlockSpec((1, 1, BLK), lambda i: (i, 0, 0)),
            pl.BlockSpec((1, C), lambda i: (0, 0)),
            pl.BlockSpec((1, C), lambda i: (0, 0)),
            pl.BlockSpec((1, C), lambda i: (0, 0)),
        ],
        out_specs=[
            pl.BlockSpec((B, C), lambda i: (0, 0)),
            pl.BlockSpec((B, C), lambda i: (0, 0)),
        ],
        out_shape=[
            jax.ShapeDtypeStruct((B, C), jnp.float32),
            jax.ShapeDtypeStruct((B, C), jnp.float32),
        ],
        scratch_shapes=[
            pltpu.VMEM((B, C), jnp.float32),
            pltpu.VMEM((B, C), jnp.float32),
            pltpu.VMEM((B, 1), jnp.float32),
        ],
    )(x, ids3, w2, b2, ms2)

    out = pl.pallas_call(
        _norm_kernel,
        grid=(NB,),
        in_specs=[
            pl.BlockSpec((BLK, C), lambda i: (i, 0)),
            pl.BlockSpec((BLK, 1), lambda i: (i, 0)),
            pl.BlockSpec((B, C), lambda i: (0, 0)),
            pl.BlockSpec((B, C), lambda i: (0, 0)),
        ],
        out_specs=pl.BlockSpec((BLK, C), lambda i: (i, 0)),
        out_shape=jax.ShapeDtypeStruct((N, C), jnp.float32),
    )(x, ids2, a_tab, b_tab)
    return out
